# baseline (speedup 1.0000x reference)
import functools

import jax
import jax.numpy as jnp
from jax import lax
from jax.experimental import pallas as pl
from jax.experimental.pallas import tpu as pltpu
from jax.experimental.pallas import tpu_sc as plsc

NPTS = 4096
KNB = 5
_NC, _NS = 2, 16
_NW = _NC * _NS
_PPW = NPTS // _NW


def _knn_body(xr_ref, xa_ref, idx_ref):
    xr = xr_ref[...]
    xa = xa_ref[...]
    d2 = lax.dot_general(xr.astype(jnp.bfloat16), xa.astype(jnp.bfloat16),
                         (((1,), (1,)), ((), ())),
                         preferred_element_type=jnp.float32)
    nr = jnp.sum(xr * xr, axis=1, keepdims=True)
    na = jnp.sum(xa * xa, axis=1)[None, :]
    p = 2.0 * d2 - nr - na
    iota = lax.broadcasted_iota(jnp.int32, p.shape, 1)
    cols = []
    for t in range(KNB):
        m = jnp.max(p, axis=1, keepdims=True)
        cand = jnp.where(p == m, iota, NPTS)
        a = jnp.min(cand, axis=1, keepdims=True)
        cols.append(a)
        if t < KNB - 1:
            p = jnp.where(iota == a, -jnp.inf, p)
    idx_ref[...] = jnp.concatenate(cols, axis=1)


def _knn(f, rblk=512):
    n, c = f.shape
    return pl.pallas_call(
        _knn_body,
        grid=(n // rblk,),
        in_specs=[pl.BlockSpec((rblk, c), lambda i: (i, 0)),
                  pl.BlockSpec((n, c), lambda i: (0, 0))],
        out_specs=pl.BlockSpec((rblk, KNB), lambda i: (i, 0)),
        out_shape=jax.ShapeDtypeStruct((n, KNB), jnp.int32),
    )(f, f)


@functools.cache
def _gather_rows_fn(c):
    mesh = plsc.VectorSubcoreMesh(core_axis_name="c", subcore_axis_name="s")

    @functools.partial(
        pl.kernel,
        out_type=jax.ShapeDtypeStruct((KNB * NPTS, c), jnp.float32),
        mesh=mesh,
        compiler_params=pltpu.CompilerParams(use_tc_tiling_on_sc=False),
        scratch_types=[
            pltpu.VMEM((KNB, _PPW), jnp.int32),
            pltpu.VMEM((KNB, _PPW, c), jnp.float32),
            pltpu.SemaphoreType.DMA,
        ],
    )
    def kern(idxt_hbm, f_hbm, fj_hbm, idx_v, rows, sem):
        wid = lax.axis_index("s") * _NC + lax.axis_index("c")
        base = wid * _PPW
        for j in range(KNB):
            pltpu.sync_copy(idxt_hbm.at[pl.ds(j * NPTS + base, _PPW)],
                            idx_v.at[j])
        cps = [pltpu.async_copy(f_hbm.at[idx_v.at[j]], rows.at[j], sem)
               for j in range(KNB)]
        for cp in cps:
            cp.wait()
        for j in range(KNB):
            pltpu.sync_copy(rows.at[j], fj_hbm.at[pl.ds(j * NPTS + base, _PPW)])

    return kern


def _edge_exact_stage1(fj3, f, w, c_real, rblk=512):
    n, cp = f.shape
    o = w.shape[0]
    nb = n // rblk

    def body(fj_ref, f_ref, w_ref, my_ref, ps_ref, pq_ref):
        fj = fj_ref[...][:, :, :c_real]
        fv = f_ref[...][:, :c_real]
        d = fj - fv[None]
        e = jnp.concatenate(
            [d, jnp.broadcast_to(fv[None], (KNB, rblk, c_real))], axis=2)
        e2 = e.reshape(KNB * rblk, 2 * c_real)
        y = lax.dot_general(e2.astype(jnp.bfloat16),
                            w_ref[...].astype(jnp.bfloat16),
                            (((1,), (1,)), ((), ())),
                            preferred_element_type=jnp.float32)
        my_ref[...] = jnp.max(y.reshape(KNB, rblk, o), axis=0)
        ps_ref[...] = jnp.sum(y, axis=0, keepdims=True)[None]
        pq_ref[...] = jnp.sum(y * y, axis=0, keepdims=True)[None]

    return pl.pallas_call(
        body,
        grid=(nb,),
        in_specs=[pl.BlockSpec((KNB, rblk, cp), lambda i: (0, i, 0)),
                  pl.BlockSpec((rblk, cp), lambda i: (i, 0)),
                  pl.BlockSpec((o, 2 * c_real), lambda i: (0, 0))],
        out_specs=[pl.BlockSpec((rblk, o), lambda i: (i, 0)),
                   pl.BlockSpec((1, 1, o), lambda i: (i, 0, 0)),
                   pl.BlockSpec((1, 1, o), lambda i: (i, 0, 0))],
        out_shape=[jax.ShapeDtypeStruct((n, o), jnp.float32),
                   jax.ShapeDtypeStruct((nb, 1, o), jnp.float32),
                   jax.ShapeDtypeStruct((nb, 1, o), jnp.float32)],
    )(fj3, f, w)


def _edge_finalize_body(my_ref, ps_ref, pq_ref, o_ref):
    my = my_ref[...]
    nk = float(NPTS * KNB)
    mean = jnp.sum(ps_ref[...], axis=0) / nk
    ey2 = jnp.sum(pq_ref[...], axis=0) / nk
    var = ey2 - mean * mean
    y = (my - mean) * lax.rsqrt(var + 1e-5)
    o_ref[...] = jnp.where(y >= 0.0, y, 0.2 * y)


def _edgeconv_exact(f, c_real, w, idxt):
    n, cp = f.shape
    fj = _gather_rows_fn(cp)(idxt.reshape(-1), f)
    fj3 = fj.reshape(KNB, n, cp)
    my, ps, pq = _edge_exact_stage1(fj3, f, w, c_real)
    return pl.pallas_call(
        _edge_finalize_body,
        out_shape=jax.ShapeDtypeStruct(my.shape, jnp.float32),
    )(my, ps, pq)


def _conv1d(parts, w):
    n = parts[0].shape[0]
    o = w.shape[0]
    nparts = len(parts)

    def body(*refs):
        w_ref = refs[nparts]
        o_ref = refs[nparts + 1]
        xcat = jnp.concatenate([refs[i][...] for i in range(nparts)], axis=1)
        y = lax.dot_general(xcat.astype(jnp.bfloat16),
                            w_ref[...].astype(jnp.bfloat16),
                            (((1,), (1,)), ((), ())),
                            preferred_element_type=jnp.float32)
        mean = jnp.sum(y, axis=0, keepdims=True) / n
        ey2 = jnp.sum(y * y, axis=0, keepdims=True) / n
        var = ey2 - mean * mean
        yv = (y - mean) * lax.rsqrt(var + 1e-5)
        o_ref[...] = jnp.where(yv >= 0.0, yv, 0.2 * yv)

    return pl.pallas_call(
        body,
        out_shape=jax.ShapeDtypeStruct((n, o), jnp.float32),
    )(*parts, w)


def kernel(x, W1s, W2s, W3s, W1c, W2c, W3c, Was, Wac, Wa):
    sem16 = jnp.pad(x[:, :10], ((0, 0), (0, 6)))
    cen16 = jnp.pad(x[:, 13:16], ((0, 0), (0, 13)))
    cen8 = cen16[:, :8]

    idx0t = jnp.transpose(_knn(cen8))

    s1 = _edgeconv_exact(sem16, 10, W1s, idx0t)
    s2 = _edgeconv_exact(s1, 64, W2s, jnp.transpose(_knn(s1)))
    s3 = _edgeconv_exact(s2, 64, W3s, jnp.transpose(_knn(s2)))
    sf = _conv1d([s1, s2, s3], Was)

    c1 = _edgeconv_exact(cen16, 3, W1c, idx0t)
    c2 = _edgeconv_exact(c1, 64, W2c, jnp.transpose(_knn(c1)))
    c3 = _edgeconv_exact(c2, 64, W3c, jnp.transpose(_knn(c2)))
    cf = _conv1d([c1, c2, c3], Wac)

    return _conv1d([sf, cf], Wa)


# speedup vs baseline: 1.0931x; 1.0931x over previous
import functools

import jax
import jax.numpy as jnp
from jax import lax
from jax.experimental import pallas as pl
from jax.experimental.pallas import tpu as pltpu
from jax.experimental.pallas import tpu_sc as plsc

NPTS = 4096
KNB = 5
_NC, _NS = 2, 16
_NW = _NC * _NS
_PPW = NPTS // _NW


def _knn_body(xr_ref, xa_ref, idx_ref):
    xr = xr_ref[...]
    xa = xa_ref[...]
    d2 = lax.dot_general(xr.astype(jnp.bfloat16), xa.astype(jnp.bfloat16),
                         (((1,), (1,)), ((), ())),
                         preferred_element_type=jnp.float32)
    nr = jnp.sum(xr * xr, axis=1, keepdims=True)
    na = jnp.sum(xa * xa, axis=1)[None, :]
    p = 2.0 * d2 - nr - na
    iota = lax.broadcasted_iota(jnp.int32, p.shape, 1)
    cols = []
    for t in range(KNB):
        a = jnp.argmax(p, axis=1, keepdims=True).astype(jnp.int32)
        cols.append(a)
        if t < KNB - 1:
            p = jnp.where(iota == a, -jnp.inf, p)
    idx_ref[...] = jnp.concatenate(cols, axis=1)


def _knn(f, rblk=512):
    n, c = f.shape
    return pl.pallas_call(
        _knn_body,
        grid=(n // rblk,),
        in_specs=[pl.BlockSpec((rblk, c), lambda i: (i, 0)),
                  pl.BlockSpec((n, c), lambda i: (0, 0))],
        out_specs=pl.BlockSpec((rblk, KNB), lambda i: (i, 0)),
        out_shape=jax.ShapeDtypeStruct((n, KNB), jnp.int32),
    )(f, f)


@functools.cache
def _gather_rows_fn(c):
    mesh = plsc.VectorSubcoreMesh(core_axis_name="c", subcore_axis_name="s")

    @functools.partial(
        pl.kernel,
        out_type=jax.ShapeDtypeStruct((KNB * NPTS, c), jnp.float32),
        mesh=mesh,
        compiler_params=pltpu.CompilerParams(use_tc_tiling_on_sc=False),
        scratch_types=[
            pltpu.VMEM((KNB, _PPW), jnp.int32),
            pltpu.VMEM((KNB, _PPW, c), jnp.float32),
            pltpu.SemaphoreType.DMA,
        ],
    )
    def kern(idxt_hbm, f_hbm, fj_hbm, idx_v, rows, sem):
        wid = lax.axis_index("s") * _NC + lax.axis_index("c")
        base = wid * _PPW
        for j in range(KNB):
            pltpu.sync_copy(idxt_hbm.at[pl.ds(j * NPTS + base, _PPW)],
                            idx_v.at[j])
        cps = [pltpu.async_copy(f_hbm.at[idx_v.at[j]], rows.at[j], sem)
               for j in range(KNB)]
        for cp in cps:
            cp.wait()
        for j in range(KNB):
            pltpu.sync_copy(rows.at[j], fj_hbm.at[pl.ds(j * NPTS + base, _PPW)])

    return kern


def _edge_exact_stage1(fj3, f, w, c_real, rblk=512):
    n, cp = f.shape
    o = w.shape[0]
    nb = n // rblk

    def body(fj_ref, f_ref, w_ref, my_ref, ps_ref, pq_ref):
        fj = fj_ref[...][:, :, :c_real]
        fv = f_ref[...][:, :c_real]
        d = fj - fv[None]
        e = jnp.concatenate(
            [d, jnp.broadcast_to(fv[None], (KNB, rblk, c_real))], axis=2)
        e2 = e.reshape(KNB * rblk, 2 * c_real)
        y = lax.dot_general(e2.astype(jnp.bfloat16),
                            w_ref[...].astype(jnp.bfloat16),
                            (((1,), (1,)), ((), ())),
                            preferred_element_type=jnp.float32)
        my_ref[...] = jnp.max(y.reshape(KNB, rblk, o), axis=0)
        ps_ref[...] = jnp.sum(y, axis=0, keepdims=True)[None]
        pq_ref[...] = jnp.sum(y * y, axis=0, keepdims=True)[None]

    return pl.pallas_call(
        body,
        grid=(nb,),
        in_specs=[pl.BlockSpec((KNB, rblk, cp), lambda i: (0, i, 0)),
                  pl.BlockSpec((rblk, cp), lambda i: (i, 0)),
                  pl.BlockSpec((o, 2 * c_real), lambda i: (0, 0))],
        out_specs=[pl.BlockSpec((rblk, o), lambda i: (i, 0)),
                   pl.BlockSpec((1, 1, o), lambda i: (i, 0, 0)),
                   pl.BlockSpec((1, 1, o), lambda i: (i, 0, 0))],
        out_shape=[jax.ShapeDtypeStruct((n, o), jnp.float32),
                   jax.ShapeDtypeStruct((nb, 1, o), jnp.float32),
                   jax.ShapeDtypeStruct((nb, 1, o), jnp.float32)],
    )(fj3, f, w)


def _edge_finalize_body(my_ref, ps_ref, pq_ref, o_ref):
    my = my_ref[...]
    nk = float(NPTS * KNB)
    mean = jnp.sum(ps_ref[...], axis=0) / nk
    ey2 = jnp.sum(pq_ref[...], axis=0) / nk
    var = ey2 - mean * mean
    y = (my - mean) * lax.rsqrt(var + 1e-5)
    o_ref[...] = jnp.where(y >= 0.0, y, 0.2 * y)


def _edgeconv_exact(f, c_real, w, idxt):
    n, cp = f.shape
    fj = _gather_rows_fn(cp)(idxt.reshape(-1), f)
    fj3 = fj.reshape(KNB, n, cp)
    my, ps, pq = _edge_exact_stage1(fj3, f, w, c_real)
    return pl.pallas_call(
        _edge_finalize_body,
        out_shape=jax.ShapeDtypeStruct(my.shape, jnp.float32),
    )(my, ps, pq)


def _conv1d(parts, w):
    n = parts[0].shape[0]
    o = w.shape[0]
    nparts = len(parts)

    def body(*refs):
        w_ref = refs[nparts]
        o_ref = refs[nparts + 1]
        xcat = jnp.concatenate([refs[i][...] for i in range(nparts)], axis=1)
        y = lax.dot_general(xcat.astype(jnp.bfloat16),
                            w_ref[...].astype(jnp.bfloat16),
                            (((1,), (1,)), ((), ())),
                            preferred_element_type=jnp.float32)
        mean = jnp.sum(y, axis=0, keepdims=True) / n
        ey2 = jnp.sum(y * y, axis=0, keepdims=True) / n
        var = ey2 - mean * mean
        yv = (y - mean) * lax.rsqrt(var + 1e-5)
        o_ref[...] = jnp.where(yv >= 0.0, yv, 0.2 * yv)

    return pl.pallas_call(
        body,
        out_shape=jax.ShapeDtypeStruct((n, o), jnp.float32),
    )(*parts, w)


def kernel(x, W1s, W2s, W3s, W1c, W2c, W3c, Was, Wac, Wa):
    sem16 = jnp.pad(x[:, :10], ((0, 0), (0, 6)))
    cen16 = jnp.pad(x[:, 13:16], ((0, 0), (0, 13)))
    cen8 = cen16[:, :8]

    idx0t = jnp.transpose(_knn(cen8))

    s1 = _edgeconv_exact(sem16, 10, W1s, idx0t)
    s2 = _edgeconv_exact(s1, 64, W2s, jnp.transpose(_knn(s1)))
    s3 = _edgeconv_exact(s2, 64, W3s, jnp.transpose(_knn(s2)))
    sf = _conv1d([s1, s2, s3], Was)

    c1 = _edgeconv_exact(cen16, 3, W1c, idx0t)
    c2 = _edgeconv_exact(c1, 64, W2c, jnp.transpose(_knn(c1)))
    c3 = _edgeconv_exact(c2, 64, W3c, jnp.transpose(_knn(c2)))
    cf = _conv1d([c1, c2, c3], Wac)

    return _conv1d([sf, cf], Wa)


# speedup vs baseline: 1.1879x; 1.0867x over previous
import functools

import jax
import jax.numpy as jnp
from jax import lax
from jax.experimental import pallas as pl
from jax.experimental.pallas import tpu as pltpu
from jax.experimental.pallas import tpu_sc as plsc

NPTS = 4096
KNB = 5
_NC, _NS = 2, 16
_NW = _NC * _NS
_PPW = NPTS // _NW


def _knn_body(xr_ref, xa_ref, idx_ref, *, c_use):
    xr = xr_ref[...][:, :c_use]
    xa = xa_ref[...][:, :c_use]
    d2 = lax.dot_general(xr.astype(jnp.bfloat16), xa.astype(jnp.bfloat16),
                         (((1,), (1,)), ((), ())),
                         preferred_element_type=jnp.float32)
    nr = jnp.sum(xr * xr, axis=1, keepdims=True)
    na = jnp.sum(xa * xa, axis=1)[None, :]
    p = 2.0 * d2 - nr - na
    iota = lax.broadcasted_iota(jnp.int32, p.shape, 1)
    cols = []
    for t in range(KNB):
        a = jnp.argmax(p, axis=1, keepdims=True).astype(jnp.int32)
        cols.append(a)
        if t < KNB - 1:
            p = jnp.where(iota == a, -jnp.inf, p)
    idx_ref[...] = jnp.concatenate(cols, axis=1)


def _knn(f, c_use, rblk=512):
    n, c = f.shape
    return pl.pallas_call(
        functools.partial(_knn_body, c_use=c_use),
        grid=(n // rblk,),
        in_specs=[pl.BlockSpec((rblk, c), lambda i: (i, 0)),
                  pl.BlockSpec((n, c), lambda i: (0, 0))],
        out_specs=pl.BlockSpec((rblk, KNB), lambda i: (i, 0)),
        out_shape=jax.ShapeDtypeStruct((n, KNB), jnp.int32),
    )(f, f)


@functools.cache
def _gather_rows_fn(c=128):
    mesh = plsc.VectorSubcoreMesh(core_axis_name="c", subcore_axis_name="s")

    @functools.partial(
        pl.kernel,
        out_type=jax.ShapeDtypeStruct((KNB * NPTS, c), jnp.float32),
        mesh=mesh,
        scratch_types=[
            pltpu.VMEM((KNB, _PPW), jnp.int32),
            pltpu.VMEM((KNB, _PPW, c), jnp.float32),
            pltpu.SemaphoreType.DMA,
        ],
    )
    def kern(idxt_hbm, f_hbm, fj_hbm, idx_v, rows, sem):
        wid = lax.axis_index("s") * _NC + lax.axis_index("c")
        base = wid * _PPW
        for j in range(KNB):
            pltpu.sync_copy(idxt_hbm.at[pl.ds(j * NPTS + base, _PPW)],
                            idx_v.at[j])
        cps = [pltpu.async_copy(f_hbm.at[idx_v.at[j]], rows.at[j], sem)
               for j in range(KNB)]
        for cp in cps:
            cp.wait()
        for j in range(KNB):
            pltpu.sync_copy(rows.at[j], fj_hbm.at[pl.ds(j * NPTS + base, _PPW)])

    return kern


def _edge_exact_stage1(fj3, f, w, c_real, rblk=512):
    n, cp = f.shape
    o = w.shape[0]
    nb = n // rblk

    def body(fj_ref, f_ref, w_ref, my_ref, ps_ref, pq_ref):
        fj = fj_ref[...][:, :, :c_real]
        fv = f_ref[...][:, :c_real]
        d = fj - fv[None]
        e = jnp.concatenate(
            [d, jnp.broadcast_to(fv[None], (KNB, rblk, c_real))], axis=2)
        e2 = e.reshape(KNB * rblk, 2 * c_real)
        y = lax.dot_general(e2.astype(jnp.bfloat16),
                            w_ref[...].astype(jnp.bfloat16),
                            (((1,), (1,)), ((), ())),
                            preferred_element_type=jnp.float32)
        my_ref[...] = jnp.max(y.reshape(KNB, rblk, o), axis=0)
        ps_ref[...] = jnp.sum(y, axis=0, keepdims=True)[None]
        pq_ref[...] = jnp.sum(y * y, axis=0, keepdims=True)[None]

    return pl.pallas_call(
        body,
        grid=(nb,),
        in_specs=[pl.BlockSpec((KNB, rblk, cp), lambda i: (0, i, 0)),
                  pl.BlockSpec((rblk, cp), lambda i: (i, 0)),
                  pl.BlockSpec((o, 2 * c_real), lambda i: (0, 0))],
        out_specs=[pl.BlockSpec((rblk, o), lambda i: (i, 0)),
                   pl.BlockSpec((1, 1, o), lambda i: (i, 0, 0)),
                   pl.BlockSpec((1, 1, o), lambda i: (i, 0, 0))],
        out_shape=[jax.ShapeDtypeStruct((n, o), jnp.float32),
                   jax.ShapeDtypeStruct((nb, 1, o), jnp.float32),
                   jax.ShapeDtypeStruct((nb, 1, o), jnp.float32)],
    )(fj3, f, w)


def _edge_finalize_body(my_ref, ps_ref, pq_ref, o_ref):
    my = my_ref[...]
    nk = float(NPTS * KNB)
    mean = jnp.sum(ps_ref[...], axis=0) / nk
    ey2 = jnp.sum(pq_ref[...], axis=0) / nk
    var = ey2 - mean * mean
    y = (my - mean) * lax.rsqrt(var + 1e-5)
    y = jnp.where(y >= 0.0, y, 0.2 * y)
    o = y.shape[1]
    opad = o_ref.shape[1]
    if opad > o:
        y = jnp.concatenate(
            [y, jnp.zeros((y.shape[0], opad - o), jnp.float32)], axis=1)
    o_ref[...] = y


def _edgeconv_exact(f, c_real, w, idxt, o_pad=None):
    n, cp = f.shape
    o = w.shape[0]
    o_pad = o_pad or o
    fj = _gather_rows_fn(cp)(idxt.reshape(-1), f)
    fj3 = fj.reshape(KNB, n, cp)
    my, ps, pq = _edge_exact_stage1(fj3, f, w, c_real)
    return pl.pallas_call(
        _edge_finalize_body,
        out_shape=jax.ShapeDtypeStruct((n, o_pad), jnp.float32),
    )(my, ps, pq)


def _conv1d(parts, creal, w):
    n = parts[0].shape[0]
    o = w.shape[0]
    nparts = len(parts)

    def body(*refs):
        w_ref = refs[nparts]
        o_ref = refs[nparts + 1]
        xcat = jnp.concatenate(
            [refs[i][...][:, :creal[i]] for i in range(nparts)], axis=1)
        y = lax.dot_general(xcat.astype(jnp.bfloat16),
                            w_ref[...].astype(jnp.bfloat16),
                            (((1,), (1,)), ((), ())),
                            preferred_element_type=jnp.float32)
        mean = jnp.sum(y, axis=0, keepdims=True) / n
        ey2 = jnp.sum(y * y, axis=0, keepdims=True) / n
        var = ey2 - mean * mean
        yv = (y - mean) * lax.rsqrt(var + 1e-5)
        o_ref[...] = jnp.where(yv >= 0.0, yv, 0.2 * yv)

    return pl.pallas_call(
        body,
        out_shape=jax.ShapeDtypeStruct((n, o), jnp.float32),
    )(*parts, w)


def kernel(x, W1s, W2s, W3s, W1c, W2c, W3c, Was, Wac, Wa):
    sem128 = jnp.pad(x[:, :10], ((0, 0), (0, 118)))
    cen128 = jnp.pad(x[:, 13:16], ((0, 0), (0, 125)))

    idx0t = jnp.transpose(_knn(cen128, 8))

    s1 = _edgeconv_exact(sem128, 10, W1s, idx0t, o_pad=128)
    s2 = _edgeconv_exact(s1, 64, W2s, jnp.transpose(_knn(s1, 64)), o_pad=128)
    s3 = _edgeconv_exact(s2, 64, W3s, jnp.transpose(_knn(s2, 64)))
    sf = _conv1d([s1, s2, s3], (64, 64, 128), Was)

    c1 = _edgeconv_exact(cen128, 3, W1c, idx0t, o_pad=128)
    c2 = _edgeconv_exact(c1, 64, W2c, jnp.transpose(_knn(c1, 64)), o_pad=128)
    c3 = _edgeconv_exact(c2, 64, W3c, jnp.transpose(_knn(c2, 64)))
    cf = _conv1d([c1, c2, c3], (64, 64, 128), Wac)

    return _conv1d([sf, cf], (128, 128), Wa)
